# baseline (speedup 1.0000x reference)
import jax
import jax.numpy as jnp
from jax.experimental import pallas as pl
from jax.experimental.pallas import tpu as pltpu

B, S, D = 2, 2048, 1024
DS = 64
N = 2048
HEADS = 6
K_TOP = 8
T_BLK = 1024
R_TILE = 32
S_ROWS = 128
LANE = 128
N_CHUNK = N // LANE

_NEG = float("-inf")

_SORT8 = (
    (0, 1), (2, 3), (4, 5), (6, 7),
    (0, 2), (1, 3), (4, 6), (5, 7),
    (1, 2), (5, 6),
    (0, 4), (1, 5), (2, 6), (3, 7),
    (2, 4), (3, 5),
    (1, 2), (3, 4), (5, 6),
)


_BITONIC8 = (
    (0, 4), (1, 5), (2, 6), (3, 7),
    (0, 2), (1, 3), (4, 6), (5, 7),
    (0, 1), (2, 3), (4, 5), (6, 7),
)


def _sort8(v):
    v = list(v)
    for i, j in _SORT8:
        lo = jnp.minimum(v[i], v[j])
        hi = jnp.maximum(v[i], v[j])
        v[i], v[j] = lo, hi
    return v


def _router_block(x_ref, w_ref, b_ref, emb_ref, o_ref, en_ref):
    t = pl.program_id(1)

    @pl.when(t == 0)
    def _normalize():
        emb = emb_ref[0]
        nrm = jnp.sqrt(jnp.sum(emb * emb, axis=1, keepdims=True)) + 1e-8
        en_ref[...] = emb / nrm

    h = jnp.dot(x_ref[...], w_ref[0],
                preferred_element_type=jnp.float32) + b_ref[0]

    def _tile(lg_slice, i, base):
        lg = lg_slice[i * R_TILE:(i + 1) * R_TILE, :]
        chunks = [lg[:, k * LANE:(k + 1) * LANE] for k in range(N_CHUNK)]
        a = _sort8(chunks[:8])
        b = _sort8(chunks[8:])
        c = [jnp.maximum(a[j], b[7 - j]) for j in range(8)]
        for p, q in _BITONIC8:
            lo = jnp.minimum(c[p], c[q])
            hi = jnp.maximum(c[p], c[q])
            c[p], c[q] = lo, hi

        m1 = jnp.max(c[7], axis=1, keepdims=True)
        thr = m1
        for j in range(2, K_TOP + 1):
            arrs = c[8 - j:]
            r = jnp.where(arrs[0] < thr, arrs[0], _NEG)
            for v in arrs[1:]:
                r = jnp.maximum(r, jnp.where(v < thr, v, _NEG))
            thr = jnp.max(r, axis=1, keepdims=True)

        e = jnp.exp(lg - m1)
        keep = jnp.where(lg >= thr, e, 0.0)
        s8 = jnp.sum(keep, axis=1, keepdims=True)
        inv = 1.0 / s8
        o_ref[0, base + i * R_TILE:base + (i + 1) * R_TILE, :] = keep * inv

    for s in range(T_BLK // S_ROWS):
        hs = h[s * S_ROWS:(s + 1) * S_ROWS]
        lgs = jax.lax.dot_general(
            hs, en_ref[...], (((1,), (1,)), ((), ())),
            preferred_element_type=jnp.float32)
        for i in range(S_ROWS // R_TILE):
            _tile(lgs, i, s * S_ROWS)


_BANK_OF_HEAD = (0, 0, 1, 2, 2, 3)


@jax.jit
def kernel(x, W_all, b_all, neuron_emb):
    xt = x.reshape(B * S, D)
    w6 = W_all.reshape(D, HEADS, DS).transpose(1, 0, 2)
    b6 = b_all.reshape(HEADS, 1, DS)
    banks = neuron_emb[:4 * N].reshape(4, N, DS)
    emb6 = banks[jnp.array(_BANK_OF_HEAD)]

    n_t = (B * S) // T_BLK
    out = pl.pallas_call(
        _router_block,
        grid=(HEADS, n_t),
        in_specs=[
            pl.BlockSpec((T_BLK, D), lambda h, t: (t, 0)),
            pl.BlockSpec((1, D, DS), lambda h, t: (h, 0, 0)),
            pl.BlockSpec((1, 1, DS), lambda h, t: (h, 0, 0)),
            pl.BlockSpec((1, N, DS), lambda h, t: (h, 0, 0)),
        ],
        out_specs=pl.BlockSpec((1, T_BLK, N), lambda h, t: (h, t, 0)),
        out_shape=jax.ShapeDtypeStruct((HEADS, B * S, N), jnp.float32),
        scratch_shapes=[pltpu.VMEM((N, DS), jnp.float32)],
    )(xt, w6, b6, emb6)
    return out.reshape(HEADS, B, S, N)


# speedup vs baseline: 1.0631x; 1.0631x over previous
import jax
import jax.numpy as jnp
from jax.experimental import pallas as pl
from jax.experimental.pallas import tpu as pltpu

B, S, D = 2, 2048, 1024
DS = 64
N = 2048
HEADS = 6
K_TOP = 8
T_BLK = 1024
R_TILE = 32
S_ROWS = 256
LANE = 128
N_CHUNK = N // LANE

_NEG = float("-inf")

_SORT8 = (
    (0, 1), (2, 3), (4, 5), (6, 7),
    (0, 2), (1, 3), (4, 6), (5, 7),
    (1, 2), (5, 6),
    (0, 4), (1, 5), (2, 6), (3, 7),
    (2, 4), (3, 5),
    (1, 2), (3, 4), (5, 6),
)


_BITONIC8 = (
    (0, 4), (1, 5), (2, 6), (3, 7),
    (0, 2), (1, 3), (4, 6), (5, 7),
    (0, 1), (2, 3), (4, 5), (6, 7),
)


def _sort8(v):
    v = list(v)
    for i, j in _SORT8:
        lo = jnp.minimum(v[i], v[j])
        hi = jnp.maximum(v[i], v[j])
        v[i], v[j] = lo, hi
    return v


def _router_block(x_ref, w_ref, b_ref, emb_ref, o_ref, en_ref):
    t = pl.program_id(1)

    @pl.when(t == 0)
    def _normalize():
        emb = emb_ref[0]
        nrm = jnp.sqrt(jnp.sum(emb * emb, axis=1, keepdims=True)) + 1e-8
        en_ref[...] = emb / nrm

    def _tile(lg_slice, i, base):
        lg = lg_slice[i * R_TILE:(i + 1) * R_TILE, :]
        chunks = [lg[:, k * LANE:(k + 1) * LANE] for k in range(N_CHUNK)]
        a = _sort8(chunks[:8])
        b = _sort8(chunks[8:])
        c = [jnp.maximum(a[j], b[7 - j]) for j in range(8)]
        for p, q in _BITONIC8:
            lo = jnp.minimum(c[p], c[q])
            hi = jnp.maximum(c[p], c[q])
            c[p], c[q] = lo, hi

        m1 = jnp.max(c[7], axis=1, keepdims=True)
        thr = m1
        for j in range(2, K_TOP + 1):
            arrs = c[8 - j:]
            r = jnp.where(arrs[0] < thr, arrs[0], _NEG)
            for v in arrs[1:]:
                r = jnp.maximum(r, jnp.where(v < thr, v, _NEG))
            thr = jnp.max(r, axis=1, keepdims=True)

        e = jnp.exp(lg - m1)
        keep = jnp.where(lg >= thr, e, 0.0)
        s8 = jnp.sum(keep, axis=1, keepdims=True)
        inv = 1.0 / s8
        o_ref[0, base + i * R_TILE:base + (i + 1) * R_TILE, :] = keep * inv

    for s in range(T_BLK // S_ROWS):
        hs = jnp.dot(x_ref[s * S_ROWS:(s + 1) * S_ROWS, :], w_ref[0],
                     preferred_element_type=jnp.float32) + b_ref[0]
        lgs = jax.lax.dot_general(
            hs, en_ref[...], (((1,), (1,)), ((), ())),
            preferred_element_type=jnp.float32)
        for i in range(S_ROWS // R_TILE):
            _tile(lgs, i, s * S_ROWS)


_BANK_OF_HEAD = (0, 0, 1, 2, 2, 3)


@jax.jit
def kernel(x, W_all, b_all, neuron_emb):
    xt = x.reshape(B * S, D)
    w6 = W_all.reshape(D, HEADS, DS).transpose(1, 0, 2)
    b6 = b_all.reshape(HEADS, 1, DS)
    banks = neuron_emb[:4 * N].reshape(4, N, DS)
    emb6 = banks[jnp.array(_BANK_OF_HEAD)]

    n_t = (B * S) // T_BLK
    out = pl.pallas_call(
        _router_block,
        grid=(HEADS, n_t),
        in_specs=[
            pl.BlockSpec((T_BLK, D), lambda h, t: (t, 0)),
            pl.BlockSpec((1, D, DS), lambda h, t: (h, 0, 0)),
            pl.BlockSpec((1, 1, DS), lambda h, t: (h, 0, 0)),
            pl.BlockSpec((1, N, DS), lambda h, t: (h, 0, 0)),
        ],
        out_specs=pl.BlockSpec((1, T_BLK, N), lambda h, t: (h, t, 0)),
        out_shape=jax.ShapeDtypeStruct((HEADS, B * S, N), jnp.float32),
        scratch_shapes=[pltpu.VMEM((N, DS), jnp.float32)],
    )(xt, w6, b6, emb6)
    return out.reshape(HEADS, B, S, N)


# speedup vs baseline: 1.0793x; 1.0153x over previous
import jax
import jax.numpy as jnp
from jax.experimental import pallas as pl
from jax.experimental.pallas import tpu as pltpu

B, S, D = 2, 2048, 1024
DS = 64
N = 2048
HEADS = 6
K_TOP = 8
T_BLK = 2048
R_TILE = 32
S_ROWS = 256
LANE = 128
N_CHUNK = N // LANE

_NEG = float("-inf")

_SORT8 = (
    (0, 1), (2, 3), (4, 5), (6, 7),
    (0, 2), (1, 3), (4, 6), (5, 7),
    (1, 2), (5, 6),
    (0, 4), (1, 5), (2, 6), (3, 7),
    (2, 4), (3, 5),
    (1, 2), (3, 4), (5, 6),
)


_BITONIC8 = (
    (0, 4), (1, 5), (2, 6), (3, 7),
    (0, 2), (1, 3), (4, 6), (5, 7),
    (0, 1), (2, 3), (4, 5), (6, 7),
)


def _sort8(v):
    v = list(v)
    for i, j in _SORT8:
        lo = jnp.minimum(v[i], v[j])
        hi = jnp.maximum(v[i], v[j])
        v[i], v[j] = lo, hi
    return v


def _router_block(x_ref, w_ref, b_ref, emb_ref, o_ref, en_ref):
    t = pl.program_id(1)

    @pl.when(t == 0)
    def _normalize():
        emb = emb_ref[0]
        nrm = jnp.sqrt(jnp.sum(emb * emb, axis=1, keepdims=True)) + 1e-8
        en_ref[...] = emb / nrm

    def _tile(lg_slice, i, base):
        lg = lg_slice[i * R_TILE:(i + 1) * R_TILE, :]
        chunks = [lg[:, k * LANE:(k + 1) * LANE] for k in range(N_CHUNK)]
        a = _sort8(chunks[:8])
        b = _sort8(chunks[8:])
        c = [jnp.maximum(a[j], b[7 - j]) for j in range(8)]
        for p, q in _BITONIC8:
            lo = jnp.minimum(c[p], c[q])
            hi = jnp.maximum(c[p], c[q])
            c[p], c[q] = lo, hi

        m1 = jnp.max(c[7], axis=1, keepdims=True)
        thr = m1
        for j in range(2, K_TOP + 1):
            arrs = c[8 - j:]
            r = jnp.where(arrs[0] < thr, arrs[0], _NEG)
            for v in arrs[1:]:
                r = jnp.maximum(r, jnp.where(v < thr, v, _NEG))
            thr = jnp.max(r, axis=1, keepdims=True)

        e = jnp.exp(lg - m1)
        keep = jnp.where(lg >= thr, e, 0.0)
        s8 = jnp.sum(keep, axis=1, keepdims=True)
        inv = 1.0 / s8
        o_ref[0, base + i * R_TILE:base + (i + 1) * R_TILE, :] = keep * inv

    for s in range(T_BLK // S_ROWS):
        hs = jnp.dot(x_ref[s * S_ROWS:(s + 1) * S_ROWS, :], w_ref[0],
                     preferred_element_type=jnp.float32) + b_ref[0]
        lgs = jax.lax.dot_general(
            hs, en_ref[...], (((1,), (1,)), ((), ())),
            preferred_element_type=jnp.float32)
        for i in range(S_ROWS // R_TILE):
            _tile(lgs, i, s * S_ROWS)


_BANK_OF_HEAD = (0, 0, 1, 2, 2, 3)


@jax.jit
def kernel(x, W_all, b_all, neuron_emb):
    xt = x.reshape(B * S, D)
    w6 = W_all.reshape(D, HEADS, DS).transpose(1, 0, 2)
    b6 = b_all.reshape(HEADS, 1, DS)
    banks = neuron_emb[:4 * N].reshape(4, N, DS)
    emb6 = banks[jnp.array(_BANK_OF_HEAD)]

    n_t = (B * S) // T_BLK
    out = pl.pallas_call(
        _router_block,
        grid=(HEADS, n_t),
        in_specs=[
            pl.BlockSpec((T_BLK, D), lambda h, t: (t, 0)),
            pl.BlockSpec((1, D, DS), lambda h, t: (h, 0, 0)),
            pl.BlockSpec((1, 1, DS), lambda h, t: (h, 0, 0)),
            pl.BlockSpec((1, N, DS), lambda h, t: (h, 0, 0)),
        ],
        out_specs=pl.BlockSpec((1, T_BLK, N), lambda h, t: (h, t, 0)),
        out_shape=jax.ShapeDtypeStruct((HEADS, B * S, N), jnp.float32),
        scratch_shapes=[pltpu.VMEM((N, DS), jnp.float32)],
        compiler_params=pltpu.CompilerParams(
            vmem_limit_bytes=64 * 1024 * 1024),
    )(xt, w6, b6, emb6)
    return out.reshape(HEADS, B, S, N)
